# baseline (speedup 1.0000x reference)
import functools

import jax
import jax.numpy as jnp
from jax import lax
from jax.experimental import pallas as pl
from jax.experimental.pallas import tpu as pltpu
from jax.experimental.pallas import tpu_sc as plsc

VOCAB = 8192
B, T = 8, 512
N = B * T
NW = 32
NSUB = 16
ROWS_PER_W = N // NW
BR = 4
NGRP = ROWS_PER_W // 16
SETW = BR * VOCAB

_mesh = plsc.VectorSubcoreMesh(core_axis_name="c", subcore_axis_name="s")


@functools.partial(
    pl.kernel,
    out_type=jax.ShapeDtypeStruct((N * VOCAB,), jnp.float32),
    mesh=_mesh,
    scratch_types=[
        pltpu.VMEM((ROWS_PER_W,), jnp.int32),
        pltpu.MemorySpace.VMEM_SHARED((NSUB, 2 * SETW), jnp.float32),
        pltpu.SemaphoreType.DMA((2,)),
        pltpu.SemaphoreType.DMA((2,)),
    ],
)
def _gather_sc(idx_hbm, emb_hbm, out_hbm, idx_v, rows_sh, gsem, ssem):
    sid = lax.axis_index("s")
    wid = sid * 2 + lax.axis_index("c")
    base = wid * ROWS_PER_W
    pltpu.sync_copy(idx_hbm.at[pl.ds(base, ROWS_PER_W)], idx_v)

    def idx_vec(g):
        return idx_v[pl.ds(g * 16, 16)]

    def gathers(vec, lane0, s):
        for j in range(BR):
            row = vec[lane0 + j]
            pltpu.async_copy(
                emb_hbm.at[pl.ds(row * VOCAB, VOCAB)],
                rows_sh.at[sid, pl.ds((s * BR + j) * VOCAB, VOCAB)],
                gsem.at[s])

    def drain_g(s):
        pltpu.make_async_copy(emb_hbm.at[pl.ds(0, SETW)],
                              rows_sh.at[sid, pl.ds(s * SETW, SETW)],
                              gsem.at[s]).wait()

    def store(c0, s):
        pltpu.async_copy(rows_sh.at[sid, pl.ds(s * SETW, SETW)],
                         out_hbm.at[pl.ds((base + c0) * VOCAB, SETW)],
                         ssem.at[s])

    def wait_s(s):
        pltpu.make_async_copy(rows_sh.at[sid, pl.ds(s * SETW, SETW)],
                              out_hbm.at[pl.ds(base * VOCAB, SETW)],
                              ssem.at[s]).wait()

    def burst(c0, s, nvec, nlane0, first=False, last=False):
        if not last:
            if not first:
                wait_s(1 - s)
            gathers(nvec, nlane0, 1 - s)
        drain_g(s)
        store(c0, s)

    v0 = idx_vec(0)
    gathers(v0, 0, 0)

    v1 = idx_vec(1)
    burst(0, 0, v0, 4, first=True)
    burst(4, 1, v0, 8)
    burst(8, 0, v0, 12)
    burst(12, 1, v1, 0)

    def body(g, carry):
        vec = idx_vec(g)
        nxt = idx_vec(g + 1)
        c0 = g * 16
        burst(c0, 0, vec, 4)
        burst(c0 + 4, 1, vec, 8)
        burst(c0 + 8, 0, vec, 12)
        burst(c0 + 12, 1, nxt, 0)
        return carry

    lax.fori_loop(1, NGRP - 1, body, 0)

    vec = idx_vec(NGRP - 1)
    c0 = (NGRP - 1) * 16
    burst(c0, 0, vec, 4)
    burst(c0 + 4, 1, vec, 8)
    burst(c0 + 8, 0, vec, 12)
    burst(c0 + 12, 1, None, 0, last=True)
    wait_s(0)
    wait_s(1)


def kernel(x, emb):
    out = _gather_sc(x.reshape(N), emb.reshape(VOCAB * VOCAB))
    return out.reshape(B, T, VOCAB)


# speedup vs baseline: 1.0014x; 1.0014x over previous
import functools

import jax
import jax.numpy as jnp
from jax import lax
from jax.experimental import pallas as pl
from jax.experimental.pallas import tpu as pltpu
from jax.experimental.pallas import tpu_sc as plsc

VOCAB = 8192
B, T = 8, 512
N = B * T
NW = 32
NSUB = 16
ROWS_PER_W = N // NW
CH = 8
NCH = ROWS_PER_W // CH
CHW = CH * VOCAB

_mesh = plsc.VectorSubcoreMesh(core_axis_name="c", subcore_axis_name="s")


@functools.partial(
    pl.kernel,
    out_type=jax.ShapeDtypeStruct((N * VOCAB,), jnp.float32),
    mesh=_mesh,
    scratch_types=[
        pltpu.VMEM((ROWS_PER_W,), jnp.int32),
        pltpu.MemorySpace.VMEM_SHARED((NSUB, CHW), jnp.float32),
        pltpu.SemaphoreType.DMA((1,)),
    ],
)
def _gather_sc(idx_hbm, emb_hbm, out_hbm, idx_v, rows_sh, gsem):
    sid = lax.axis_index("s")
    wid = sid * 2 + lax.axis_index("c")
    base = wid * ROWS_PER_W
    pltpu.sync_copy(idx_hbm.at[pl.ds(base, ROWS_PER_W)], idx_v)

    def group(g, carry):
        v = idx_v[pl.ds(g * 16, 16)]
        for h in range(2):
            c0 = g * 16 + h * CH
            for j in range(CH):
                row = v[h * CH + j]
                pltpu.async_copy(
                    emb_hbm.at[pl.ds(row * VOCAB, VOCAB)],
                    rows_sh.at[sid, pl.ds(j * VOCAB, VOCAB)],
                    gsem.at[0])
            pltpu.make_async_copy(emb_hbm.at[pl.ds(0, CHW)],
                                  rows_sh.at[sid],
                                  gsem.at[0]).wait()
            pltpu.sync_copy(rows_sh.at[sid],
                            out_hbm.at[pl.ds((base + c0) * VOCAB, CHW)])
        return carry

    lax.fori_loop(0, ROWS_PER_W // 16, group, 0)


def kernel(x, emb):
    out = _gather_sc(x.reshape(N), emb.reshape(VOCAB * VOCAB))
    return out.reshape(B, T, VOCAB)


# speedup vs baseline: 3.5129x; 3.5078x over previous
import functools

import jax
import jax.numpy as jnp
from jax import lax
from jax.experimental import pallas as pl
from jax.experimental.pallas import tpu as pltpu
from jax.experimental.pallas import tpu_sc as plsc

VOCAB = 8192
B, T = 8, 512
N = B * T
NW = 32
ROWS_PER_W = N // NW
CH = 8
NCH = ROWS_PER_W // CH

_mesh = plsc.VectorSubcoreMesh(core_axis_name="c", subcore_axis_name="s")


@functools.partial(
    pl.kernel,
    out_type=jax.ShapeDtypeStruct((N, VOCAB), jnp.float32),
    mesh=_mesh,
    scratch_types=[
        pltpu.VMEM((ROWS_PER_W,), jnp.int32),
        pltpu.VMEM((CH, VOCAB), jnp.float32),
        pltpu.SemaphoreType.DMA,
    ],
)
def _gather_sc(idx_hbm, emb_hbm, out_hbm, idx_v, rows_v, sem):
    wid = lax.axis_index("s") * 2 + lax.axis_index("c")
    base = wid * ROWS_PER_W
    pltpu.sync_copy(idx_hbm.at[pl.ds(base, ROWS_PER_W)], idx_v)

    def chunk(c, carry):
        c0 = c * CH
        pltpu.async_copy(emb_hbm.at[idx_v.at[pl.ds(c0, CH)]], rows_v,
                         sem).wait()
        pltpu.sync_copy(rows_v, out_hbm.at[pl.ds(base + c0, CH)])
        return carry

    lax.fori_loop(0, NCH, chunk, 0)


def kernel(x, emb):
    return _gather_sc(x.reshape(N), emb).reshape(B, T, VOCAB)


# speedup vs baseline: 3.8490x; 1.0957x over previous
import functools

import jax
import jax.numpy as jnp
from jax import lax
from jax.experimental import pallas as pl
from jax.experimental.pallas import tpu as pltpu
from jax.experimental.pallas import tpu_sc as plsc

VOCAB = 8192
B, T = 8, 512
N = B * T
NW = 32
ROWS_PER_W = N // NW
CH = 4
NCH = ROWS_PER_W // CH
NG = NCH // 2

_mesh = plsc.VectorSubcoreMesh(core_axis_name="c", subcore_axis_name="s")


@functools.partial(
    pl.kernel,
    out_type=jax.ShapeDtypeStruct((N, VOCAB), jnp.float32),
    mesh=_mesh,
    scratch_types=[
        pltpu.VMEM((2 * ROWS_PER_W,), jnp.int32),
        pltpu.VMEM((2, CH, VOCAB), jnp.float32),
        pltpu.SemaphoreType.DMA((2,)),
        pltpu.SemaphoreType.DMA((2,)),
    ],
)
def _gather_sc(idx_hbm, emb_hbm, out_hbm, idx_v, rows_v, gsem, ssem):
    wid = lax.axis_index("s") * 2 + lax.axis_index("c")
    base = wid * ROWS_PER_W
    pltpu.sync_copy(idx_hbm.at[pl.ds(2 * base, 2 * ROWS_PER_W)], idx_v)

    def gather(c, b):
        pltpu.async_copy(emb_hbm.at[idx_v.at[pl.ds(c * 8, CH)]],
                         rows_v.at[b], gsem.at[b])

    def wait_g(b):
        pltpu.make_async_copy(emb_hbm.at[pl.ds(0, CH)], rows_v.at[b],
                              gsem.at[b]).wait()

    def store(c, b):
        pltpu.async_copy(rows_v.at[b],
                         out_hbm.at[pl.ds(base + c * CH, CH)], ssem.at[b])

    def wait_s(b):
        pltpu.make_async_copy(rows_v.at[b], out_hbm.at[pl.ds(base, CH)],
                              ssem.at[b]).wait()

    gather(0, 0)
    gather(1, 1)
    wait_g(0)
    store(0, 0)
    wait_s(0)
    gather(2, 0)
    wait_g(1)
    store(1, 1)

    def body(g, carry):
        c = g * 2
        wait_s(1)
        gather(c + 1, 1)
        wait_g(0)
        store(c, 0)
        wait_s(0)
        gather(c + 2, 0)
        wait_g(1)
        store(c + 1, 1)
        return carry

    lax.fori_loop(1, NG - 1, body, 0)

    c = NCH - 2
    wait_s(1)
    gather(c + 1, 1)
    wait_g(0)
    store(c, 0)
    wait_g(1)
    store(c + 1, 1)
    wait_s(0)
    wait_s(1)


def kernel(x, emb):
    xc = x.reshape(N // CH, CH)
    xp = jnp.concatenate([xc, jnp.zeros_like(xc)], axis=1).reshape(2 * N)
    return _gather_sc(xp, emb).reshape(B, T, VOCAB)


# speedup vs baseline: 3.8666x; 1.0046x over previous
import functools

import jax
import jax.numpy as jnp
from jax import lax
from jax.experimental import pallas as pl
from jax.experimental.pallas import tpu as pltpu
from jax.experimental.pallas import tpu_sc as plsc

VOCAB = 8192
B, T = 8, 512
N = B * T
NW = 32
ROWS_PER_W = N // NW
CH = 4
NCH = ROWS_PER_W // CH

_mesh = plsc.VectorSubcoreMesh(core_axis_name="c", subcore_axis_name="s")


@functools.partial(
    pl.kernel,
    out_type=jax.ShapeDtypeStruct((N, VOCAB), jnp.float32),
    mesh=_mesh,
    scratch_types=[
        pltpu.VMEM((2 * ROWS_PER_W,), jnp.int32),
        pltpu.VMEM((3, CH, VOCAB), jnp.float32),
        pltpu.SemaphoreType.DMA((3,)),
        pltpu.SemaphoreType.DMA((3,)),
    ],
)
def _gather_sc(idx_hbm, emb_hbm, out_hbm, idx_v, rows_v, gsem, ssem):
    wid = lax.axis_index("s") * 2 + lax.axis_index("c")
    base = wid * ROWS_PER_W
    pltpu.sync_copy(idx_hbm.at[pl.ds(2 * base, 2 * ROWS_PER_W)], idx_v)

    def gather(c, b):
        pltpu.async_copy(emb_hbm.at[idx_v.at[pl.ds(c * 8, CH)]],
                         rows_v.at[b], gsem.at[b])

    def wait_g(b):
        pltpu.make_async_copy(emb_hbm.at[pl.ds(0, CH)], rows_v.at[b],
                              gsem.at[b]).wait()

    def store(c, b):
        pltpu.async_copy(rows_v.at[b],
                         out_hbm.at[pl.ds(base + c * CH, CH)], ssem.at[b])

    def wait_s(b):
        pltpu.make_async_copy(rows_v.at[b], out_hbm.at[pl.ds(base, CH)],
                              ssem.at[b]).wait()

    gather(0, 0)
    gather(1, 1)
    gather(2, 2)
    wait_g(0)
    store(0, 0)
    wait_s(0)
    gather(3, 0)
    wait_g(1)
    store(1, 1)
    wait_s(1)
    gather(4, 1)
    wait_g(2)
    store(2, 2)

    def body(g, carry):
        c = g * 3
        for k in range(3):
            ck = c + k
            wait_s((ck + 2) % 3)
            gather_c = ck + 2
            pltpu.async_copy(
                emb_hbm.at[idx_v.at[pl.ds(gather_c * 8, CH)]],
                rows_v.at[(k + 2) % 3], gsem.at[(k + 2) % 3])
            wait_g(k)
            store(ck, k)
        return carry

    lax.fori_loop(1, NCH // 3, body, 0)

    wait_g(0)
    store(NCH - 2, 0)
    wait_g(1)
    store(NCH - 1, 1)
    wait_s(2)
    wait_s(0)
    wait_s(1)


def kernel(x, emb):
    xc = x.reshape(N // CH, CH)
    xp = jnp.concatenate([xc, jnp.zeros_like(xc)], axis=1).reshape(2 * N)
    return _gather_sc(xp, emb).reshape(B, T, VOCAB)
